# baseline (speedup 1.0000x reference)
import functools

import jax
import jax.numpy as jnp
from jax import lax
from jax.experimental import pallas as pl
from jax.experimental.pallas import tpu as pltpu
from jax.experimental.pallas import tpu_sc as plsc

DIM = 128
MARGIN = 2.0
NC, NS = 2, 16
NW = NC * NS
CHUNK = 32
LANES = 16
KREG = DIM // LANES


def _allsum(v, perms):
    for p in perms:
        v = v + v.at[p].get(mode="promise_in_bounds")
    return v


def _rsqrt_f32(x):
    xb = lax.bitcast_convert_type(x, jnp.int32)
    yb = jnp.int32(0x5F3759DF) - lax.shift_right_logical(xb, 1)
    y = lax.bitcast_convert_type(yb, jnp.float32)
    hx = x * jnp.float32(0.5)
    for _ in range(2):
        y = y * (jnp.float32(1.5) - hx * y * y)
    return y


def _make_sc_kernel(B, n_rel):
    assert B % (NW * CHUNK) == 0
    per_w = B // NW
    n_chunks = per_w // CHUNK
    mesh = plsc.VectorSubcoreMesh(core_axis_name="c", subcore_axis_name="s")
    rowbuf = pltpu.VMEM((2, CHUNK, DIM), jnp.float32)

    @functools.partial(
        pl.kernel,
        mesh=mesh,
        out_type=jax.ShapeDtypeStruct((B,), jnp.float32),
        scratch_types=[
            pltpu.VMEM((per_w,), jnp.int32),
            pltpu.VMEM((per_w,), jnp.int32),
            pltpu.VMEM((per_w,), jnp.int32),
            [rowbuf] * 6,
            pltpu.VMEM((per_w,), jnp.float32),
            pltpu.VMEM_SHARED((n_rel, DIM), jnp.float32),
            pltpu.VMEM_SHARED((n_rel, DIM), jnp.float32),
            pltpu.SemaphoreType.DMA,
        ],
    )
    def sc_kernel(hs_h, rs_h, ts_h, ee_h, re_h, et_h, rt_h, out_h,
                  hidx, ridx, tidx, bufs, outv, re_s, rt_s, sem):
        hbuf, tbuf, rbuf, htrb, ttrb, rtrb = bufs
        wid = lax.axis_index("s") * NC + lax.axis_index("c")
        base_w = wid * per_w
        lane_iota = lax.iota(jnp.int32, LANES)
        perms = [lax.bitwise_xor(lane_iota, jnp.int32(s)) for s in (8, 4, 2, 1)]

        @pl.when(lax.axis_index("s") == 0)
        def _():
            pltpu.sync_copy(re_h, re_s)
            pltpu.sync_copy(rt_h, rt_s)

        pltpu.sync_copy(hs_h.at[pl.ds(base_w, per_w)], hidx)
        pltpu.sync_copy(rs_h.at[pl.ds(base_w, per_w)], ridx)
        pltpu.sync_copy(ts_h.at[pl.ds(base_w, per_w)], tidx)
        plsc.subcore_barrier()

        def copies(c, p):
            hi = hidx.at[pl.ds(c * CHUNK, CHUNK)]
            ri = ridx.at[pl.ds(c * CHUNK, CHUNK)]
            ti = tidx.at[pl.ds(c * CHUNK, CHUNK)]
            return [
                pltpu.make_async_copy(ee_h.at[hi], hbuf.at[p], sem),
                pltpu.make_async_copy(ee_h.at[ti], tbuf.at[p], sem),
                pltpu.make_async_copy(re_h.at[ri], rbuf.at[p], sem),
                pltpu.make_async_copy(et_h.at[hi], htrb.at[p], sem),
                pltpu.make_async_copy(et_h.at[ti], ttrb.at[p], sem),
                pltpu.make_async_copy(rt_h.at[ri], rtrb.at[p], sem),
            ]

        def start6(c, p):
            for cp in copies(c, p):
                cp.start()

        def wait6(c, p):
            for cp in copies(c, p):
                cp.wait()

        def tri_score(p, i):
            acc_h = jnp.zeros((LANES,), jnp.float32)
            acc_t = jnp.zeros((LANES,), jnp.float32)
            hk, tk = [], []
            for k in range(KREG):
                h = hbuf[p, i, pl.ds(k * LANES, LANES)]
                ht = htrb[p, i, pl.ds(k * LANES, LANES)]
                t = tbuf[p, i, pl.ds(k * LANES, LANES)]
                tt = ttrb[p, i, pl.ds(k * LANES, LANES)]
                acc_h = acc_h + h * ht
                acc_t = acc_t + t * tt
                hk.append(h)
                tk.append(t)
            s_h = _allsum(acc_h, perms)
            s_t = _allsum(acc_t, perms)
            nh = jnp.zeros((LANES,), jnp.float32)
            nt = jnp.zeros((LANES,), jnp.float32)
            hp, tp = [], []
            for k in range(KREG):
                rt = rtrb[p, i, pl.ds(k * LANES, LANES)]
                hpk = hk[k] + s_h * rt
                tpk = tk[k] + s_t * rt
                nh = nh + hpk * hpk
                nt = nt + tpk * tpk
                hp.append(hpk)
                tp.append(tpk)
            inv_h = _rsqrt_f32(jnp.maximum(_allsum(nh, perms), jnp.float32(1e-24)))
            inv_t = _rsqrt_f32(jnp.maximum(_allsum(nt, perms), jnp.float32(1e-24)))
            acc = jnp.zeros((LANES,), jnp.float32)
            for k in range(KREG):
                r = rbuf[p, i, pl.ds(k * LANES, LANES)]
                acc = acc + jnp.abs(hp[k] * inv_h + r - tp[k] * inv_t)
            return jnp.float32(MARGIN) - _allsum(acc, perms)

        start6(0, 0)

        def chunk_body(ci, _):
            p = lax.rem(ci, 2)
            wait6(ci, p)

            @pl.when(ci + 1 < n_chunks)
            def _():
                start6(ci + 1, 1 - p)

            def group_body(g, _):
                scores = jnp.zeros((LANES,), jnp.float32)
                for j in range(LANES):
                    res = tri_score(p, g * LANES + j)
                    scores = jnp.where(lane_iota == j, res, scores)
                outv[pl.ds(ci * CHUNK + g * LANES, LANES)] = scores
                return 0

            lax.fori_loop(0, CHUNK // LANES, group_body, 0)
            return 0

        lax.fori_loop(0, n_chunks, chunk_body, 0)
        pltpu.sync_copy(outv, out_h.at[pl.ds(base_w, per_w)])

    return sc_kernel


def kernel(hs, rs, ts, ent_embs, rel_embs, ent_transfer, rel_transfer):
    B = hs.shape[0]
    hs = hs.astype(jnp.int32)
    rs = rs.astype(jnp.int32)
    ts = ts.astype(jnp.int32)
    return _make_sc_kernel(B, rel_embs.shape[0])(hs, rs, ts, ent_embs, rel_embs,
                                                 ent_transfer, rel_transfer)


# speedup vs baseline: 1.0811x; 1.0811x over previous
import functools

import jax
import jax.numpy as jnp
from jax import lax
from jax.experimental import pallas as pl
from jax.experimental.pallas import tpu as pltpu
from jax.experimental.pallas import tpu_sc as plsc

DIM = 128
MARGIN = 2.0
NC, NS = 2, 16
NW = NC * NS
CHUNK = 64
SMALL = 32
LANES = 16
KREG = DIM // LANES


def _allsum(v, perms):
    for p in perms:
        v = v + v.at[p].get(mode="promise_in_bounds")
    return v


def _rsqrt_f32(x):
    xb = lax.bitcast_convert_type(x, jnp.int32)
    yb = jnp.int32(0x5F3759DF) - lax.shift_right_logical(xb, 1)
    y = lax.bitcast_convert_type(yb, jnp.float32)
    hx = x * jnp.float32(0.5)
    for _ in range(2):
        y = y * (jnp.float32(1.5) - hx * y * y)
    return y


def _make_sc_kernel(B):
    assert B % (NW * CHUNK) == 0
    per_w = B // NW
    n_chunks = (per_w - 2 * SMALL) // CHUNK + 2
    assert 2 * SMALL + (n_chunks - 2) * CHUNK == per_w
    mesh = plsc.VectorSubcoreMesh(core_axis_name="c", subcore_axis_name="s")
    rowbuf = pltpu.VMEM((2, CHUNK, DIM), jnp.float32)

    @functools.partial(
        pl.kernel,
        mesh=mesh,
        out_type=jax.ShapeDtypeStruct((B,), jnp.float32),
        scratch_types=[
            pltpu.VMEM((per_w,), jnp.int32),
            pltpu.VMEM((per_w,), jnp.int32),
            pltpu.VMEM((per_w,), jnp.int32),
            [rowbuf] * 6,
            pltpu.VMEM((per_w,), jnp.float32),
            pltpu.SemaphoreType.DMA,
        ],
    )
    def sc_kernel(hs_h, rs_h, ts_h, ee_h, re_h, et_h, rt_h, out_h,
                  hidx, ridx, tidx, bufs, outv, sem):
        hbuf, tbuf, rbuf, htrb, ttrb, rtrb = bufs
        wid = lax.axis_index("s") * NC + lax.axis_index("c")
        base_w = wid * per_w
        lane_iota = lax.iota(jnp.int32, LANES)
        perms = [lax.bitwise_xor(lane_iota, jnp.int32(s)) for s in (8, 4, 2, 1)]

        pltpu.sync_copy(hs_h.at[pl.ds(base_w, per_w)], hidx)
        pltpu.sync_copy(rs_h.at[pl.ds(base_w, per_w)], ridx)
        pltpu.sync_copy(ts_h.at[pl.ds(base_w, per_w)], tidx)

        def chunk_off(ci):
            return pl.multiple_of(jnp.maximum(0, CHUNK * ci - (CHUNK - SMALL)),
                                  SMALL)

        def copies(ci, p, size):
            off = chunk_off(ci)
            hi = hidx.at[pl.ds(off, size)]
            ri = ridx.at[pl.ds(off, size)]
            ti = tidx.at[pl.ds(off, size)]
            dst = lambda b: b.at[p, pl.ds(0, size)]
            return [
                pltpu.make_async_copy(ee_h.at[hi], dst(hbuf), sem),
                pltpu.make_async_copy(ee_h.at[ti], dst(tbuf), sem),
                pltpu.make_async_copy(re_h.at[ri], dst(rbuf), sem),
                pltpu.make_async_copy(et_h.at[hi], dst(htrb), sem),
                pltpu.make_async_copy(et_h.at[ti], dst(ttrb), sem),
                pltpu.make_async_copy(rt_h.at[ri], dst(rtrb), sem),
            ]

        def start6(ci, p):
            small = jnp.logical_or(ci == 0, ci == n_chunks - 1)

            @pl.when(small)
            def _():
                for cp in copies(ci, p, SMALL):
                    cp.start()

            @pl.when(jnp.logical_not(small))
            def _():
                for cp in copies(ci, p, CHUNK):
                    cp.start()

        def wait6(ci, p):
            small = jnp.logical_or(ci == 0, ci == n_chunks - 1)

            @pl.when(small)
            def _():
                for cp in copies(ci, p, SMALL):
                    cp.wait()

            @pl.when(jnp.logical_not(small))
            def _():
                for cp in copies(ci, p, CHUNK):
                    cp.wait()

        def tri_score(p, i):
            acc_h = jnp.zeros((LANES,), jnp.float32)
            acc_t = jnp.zeros((LANES,), jnp.float32)
            hk, tk = [], []
            for k in range(KREG):
                h = hbuf[p, i, pl.ds(k * LANES, LANES)]
                ht = htrb[p, i, pl.ds(k * LANES, LANES)]
                t = tbuf[p, i, pl.ds(k * LANES, LANES)]
                tt = ttrb[p, i, pl.ds(k * LANES, LANES)]
                acc_h = acc_h + h * ht
                acc_t = acc_t + t * tt
                hk.append(h)
                tk.append(t)
            s_h = _allsum(acc_h, perms)
            s_t = _allsum(acc_t, perms)
            nh = jnp.zeros((LANES,), jnp.float32)
            nt = jnp.zeros((LANES,), jnp.float32)
            hp, tp = [], []
            for k in range(KREG):
                rt = rtrb[p, i, pl.ds(k * LANES, LANES)]
                hpk = hk[k] + s_h * rt
                tpk = tk[k] + s_t * rt
                nh = nh + hpk * hpk
                nt = nt + tpk * tpk
                hp.append(hpk)
                tp.append(tpk)
            inv_h = _rsqrt_f32(jnp.maximum(_allsum(nh, perms), jnp.float32(1e-24)))
            inv_t = _rsqrt_f32(jnp.maximum(_allsum(nt, perms), jnp.float32(1e-24)))
            acc = jnp.zeros((LANES,), jnp.float32)
            for k in range(KREG):
                r = rbuf[p, i, pl.ds(k * LANES, LANES)]
                acc = acc + jnp.abs(hp[k] * inv_h + r - tp[k] * inv_t)
            return jnp.float32(MARGIN) - _allsum(acc, perms)

        start6(0, 0)

        def chunk_body(ci, _):
            p = lax.rem(ci, 2)
            wait6(ci, p)

            @pl.when(ci + 1 < n_chunks)
            def _():
                start6(ci + 1, 1 - p)

            off = chunk_off(ci)
            n_groups = jnp.where(
                jnp.logical_or(ci == 0, ci == n_chunks - 1),
                SMALL // LANES, CHUNK // LANES)

            def group_body(g, _):
                scores = jnp.zeros((LANES,), jnp.float32)
                for j in range(LANES):
                    res = tri_score(p, g * LANES + j)
                    scores = jnp.where(lane_iota == j, res, scores)
                outv[pl.ds(off + g * LANES, LANES)] = scores
                return 0

            lax.fori_loop(0, n_groups, group_body, 0)
            return 0

        lax.fori_loop(0, n_chunks, chunk_body, 0)
        pltpu.sync_copy(outv, out_h.at[pl.ds(base_w, per_w)])

    return sc_kernel


def kernel(hs, rs, ts, ent_embs, rel_embs, ent_transfer, rel_transfer):
    B = hs.shape[0]
    hs = hs.astype(jnp.int32)
    rs = rs.astype(jnp.int32)
    ts = ts.astype(jnp.int32)
    return _make_sc_kernel(B)(hs, rs, ts, ent_embs, rel_embs,
                              ent_transfer, rel_transfer)


# speedup vs baseline: 1.1332x; 1.0482x over previous
import functools

import jax
import jax.numpy as jnp
from jax import lax
from jax.experimental import pallas as pl
from jax.experimental.pallas import tpu as pltpu
from jax.experimental.pallas import tpu_sc as plsc

DIM = 128
MARGIN = 2.0
NC, NS = 2, 16
NW = NC * NS
CHUNK = 64
LANES = 16
KREG = DIM // LANES


def _rsqrt_f32(x):
    xb = lax.bitcast_convert_type(x, jnp.int32)
    yb = jnp.int32(0x5F3759DF) - lax.shift_right_logical(xb, 1)
    y = lax.bitcast_convert_type(yb, jnp.float32)
    hx = x * jnp.float32(0.5)
    for _ in range(2):
        y = y * (jnp.float32(1.5) - hx * y * y)
    return y


def _make_sc_kernel(B):
    assert B % (NW * CHUNK) == 0
    per_w = B // NW
    n_chunks = per_w // CHUNK
    mesh = plsc.VectorSubcoreMesh(core_axis_name="c", subcore_axis_name="s")
    rowbuf = pltpu.VMEM((2, CHUNK, DIM), jnp.float32)

    @functools.partial(
        pl.kernel,
        mesh=mesh,
        out_type=jax.ShapeDtypeStruct((B,), jnp.float32),
        scratch_types=[
            pltpu.VMEM((per_w,), jnp.int32),
            pltpu.VMEM((per_w,), jnp.int32),
            pltpu.VMEM((per_w,), jnp.int32),
            [rowbuf] * 6,
            pltpu.VMEM((per_w,), jnp.float32),
            pltpu.SemaphoreType.DMA,
        ],
    )
    def sc_kernel(hs_h, rs_h, ts_h, ee_h, re_h, et_h, rt_h, out_h,
                  hidx, ridx, tidx, bufs, outv, sem):
        hbuf, tbuf, rbuf, htrb, ttrb, rtrb = bufs
        wid = lax.axis_index("s") * NC + lax.axis_index("c")
        base_w = wid * per_w
        lane = lax.iota(jnp.int32, LANES)
        p8, p4, p2, p1 = (lax.bitwise_xor(lane, jnp.int32(s)) for s in (8, 4, 2, 1))
        idx0 = lax.bitwise_and(lane, jnp.int32(0))
        idx8 = lax.bitwise_or(idx0, jnp.int32(8))
        lo_half = lane < jnp.int32(8)
        bit_masks = [lax.bitwise_and(lane, jnp.int32(b)) == 0 for b in (8, 4, 2, 1)]
        brev = lax.bitwise_or(
            lax.bitwise_or(lax.shift_left(lax.bitwise_and(lane, 1), 3),
                           lax.shift_left(lax.bitwise_and(lane, 2), 1)),
            lax.bitwise_or(lax.shift_right_logical(lax.bitwise_and(lane, 4), 1),
                           lax.shift_right_logical(lax.bitwise_and(lane, 8), 3)))

        def gperm(v, p):
            return v.at[p].get(mode="promise_in_bounds")

        def dual_allsum(va, vb):
            m = jnp.where(lo_half, va + gperm(va, p8), vb + gperm(vb, p8))
            for p in (p4, p2, p1):
                m = m + gperm(m, p)
            return m

        pltpu.sync_copy(hs_h.at[pl.ds(base_w, per_w)], hidx)
        pltpu.sync_copy(rs_h.at[pl.ds(base_w, per_w)], ridx)
        pltpu.sync_copy(ts_h.at[pl.ds(base_w, per_w)], tidx)

        def copies(c, p):
            hi = hidx.at[pl.ds(c * CHUNK, CHUNK)]
            ri = ridx.at[pl.ds(c * CHUNK, CHUNK)]
            ti = tidx.at[pl.ds(c * CHUNK, CHUNK)]
            return [
                pltpu.make_async_copy(ee_h.at[hi], hbuf.at[p], sem),
                pltpu.make_async_copy(ee_h.at[ti], tbuf.at[p], sem),
                pltpu.make_async_copy(re_h.at[ri], rbuf.at[p], sem),
                pltpu.make_async_copy(et_h.at[hi], htrb.at[p], sem),
                pltpu.make_async_copy(et_h.at[ti], ttrb.at[p], sem),
                pltpu.make_async_copy(rt_h.at[ri], rtrb.at[p], sem),
            ]

        def start6(c, p):
            for cp in copies(c, p):
                cp.start()

        def wait6(c, p):
            for cp in copies(c, p):
                cp.wait()

        def tri_acc(p, i):
            acc_h = jnp.zeros((LANES,), jnp.float32)
            acc_t = jnp.zeros((LANES,), jnp.float32)
            hk, tk = [], []
            for k in range(KREG):
                h = hbuf[p, i, pl.ds(k * LANES, LANES)]
                ht = htrb[p, i, pl.ds(k * LANES, LANES)]
                t = tbuf[p, i, pl.ds(k * LANES, LANES)]
                tt = ttrb[p, i, pl.ds(k * LANES, LANES)]
                acc_h = acc_h + h * ht
                acc_t = acc_t + t * tt
                hk.append(h)
                tk.append(t)
            s_ht = dual_allsum(acc_h, acc_t)
            s_h = gperm(s_ht, idx0)
            s_t = gperm(s_ht, idx8)
            nh = jnp.zeros((LANES,), jnp.float32)
            nt = jnp.zeros((LANES,), jnp.float32)
            hp, tp = [], []
            for k in range(KREG):
                rt = rtrb[p, i, pl.ds(k * LANES, LANES)]
                hpk = hk[k] + s_h * rt
                tpk = tk[k] + s_t * rt
                nh = nh + hpk * hpk
                nt = nt + tpk * tpk
                hp.append(hpk)
                tp.append(tpk)
            n_ht = dual_allsum(nh, nt)
            inv = _rsqrt_f32(jnp.maximum(n_ht, jnp.float32(1e-24)))
            inv_h = gperm(inv, idx0)
            inv_t = gperm(inv, idx8)
            acc = jnp.zeros((LANES,), jnp.float32)
            for k in range(KREG):
                r = rbuf[p, i, pl.ds(k * LANES, LANES)]
                acc = acc + jnp.abs(hp[k] * inv_h + r - tp[k] * inv_t)
            return acc

        start6(0, 0)

        def chunk_body(ci, _):
            p = lax.rem(ci, 2)
            wait6(ci, p)

            @pl.when(ci + 1 < n_chunks)
            def _():
                start6(ci + 1, 1 - p)

            @plsc.parallel_loop(0, CHUNK // LANES)
            def group_body(g):
                vecs = [tri_acc(p, g * LANES + j) for j in range(LANES)]
                for pp, mask in zip((p8, p4, p2, p1), bit_masks):
                    vecs = [
                        jnp.where(mask, a + gperm(a, pp), b + gperm(b, pp))
                        for a, b in zip(vecs[0::2], vecs[1::2])
                    ]
                scores = jnp.float32(MARGIN) - gperm(vecs[0], brev)
                outv[pl.ds(ci * CHUNK + g * LANES, LANES)] = scores

            return 0

        lax.fori_loop(0, n_chunks, chunk_body, 0)
        pltpu.sync_copy(outv, out_h.at[pl.ds(base_w, per_w)])

    return sc_kernel


def kernel(hs, rs, ts, ent_embs, rel_embs, ent_transfer, rel_transfer):
    B = hs.shape[0]
    hs = hs.astype(jnp.int32)
    rs = rs.astype(jnp.int32)
    ts = ts.astype(jnp.int32)
    return _make_sc_kernel(B)(hs, rs, ts, ent_embs, rel_embs,
                              ent_transfer, rel_transfer)
